# baseline (speedup 1.0000x reference)
import jax
import jax.numpy as jnp
from jax.experimental import pallas as pl
from jax.experimental.pallas import tpu as pltpu

_B, _C, _N = 64, 1024, 256
_NHID, _NCLASS, _K = 1024, 1024, 10


def _dot(a, b, dims, prec):
    return jax.lax.dot_general(a, b, dims,
                               preferred_element_type=jnp.float32,
                               precision=prec)


def _fold_body(w1_ref, g_w_ref, g_b_ref, m_ref, c1_ref):
    prec = jax.lax.Precision.DEFAULT
    m_ref[...] = _dot(w1_ref[...], g_w_ref[...],
                      (((0,), (0,)), ((), ())), prec).astype(jnp.bfloat16)
    c1_ref[...] = _dot(w1_ref[...], g_b_ref[...],
                       (((0,), (0,)), ((), ())), prec)


def _topk_iters(cur, n_iters, ts):
    neg = jnp.float32(-jnp.inf)
    for _ in range(n_iters):
        t = jnp.max(cur, axis=0, keepdims=True)
        cur = jnp.where(cur == t, neg, cur)
        ts.append(t)
    return cur


def _slow_tie_state(A):
    f32 = jnp.float32
    neg = jnp.float32(-jnp.inf)
    cur = A
    total = jnp.zeros((1, _N), f32)
    thresh = jnp.zeros((1, _N), f32)
    keep_at = jnp.zeros((1, _N), f32)
    for _ in range(_K):
        t = jnp.max(cur, axis=0, keepdims=True)
        c = jnp.sum((cur == t).astype(f32), axis=0, keepdims=True)
        done_now = jnp.logical_and(total < _K, total + c >= _K)
        thresh = jnp.where(done_now, t, thresh)
        keep_at = jnp.where(done_now, _K - total, keep_at)
        total = total + c
        cur = jnp.where(cur == t, neg, cur)
    return thresh, keep_at


def _gcn_body(vc_ref, vp_ref, m_ref, c1_ref, b1_ref, w2_ref, b2_ref,
              out_ref, scr_ref):
    f32 = jnp.float32
    prec = jax.lax.Precision.DEFAULT
    bf16 = jnp.bfloat16
    b = pl.program_id(0)
    slot = jax.lax.rem(b, 2)

    v_c = vc_ref[0]
    v_p = vp_ref[0]
    ah16 = scr_ref[jax.lax.rem(b + 1, 2)]

    A = _dot(v_c, v_c, (((0,), (0,)), ((), ())), prec)
    h1 = _dot(m_ref[...], v_p.astype(bf16),
              (((1,), (0,)), ((), ())), prec) + c1_ref[...]

    lo = A[:_N // 2, :]
    hi = A[_N // 2:, :]
    ts = []
    lo = _topk_iters(lo, 4, ts)
    hi = _topk_iters(hi, 4, ts)

    x = jnp.maximum(
        _dot(h1.astype(bf16), ah16, (((1,), (0,)), ((), ())), prec)
        + b1_ref[...], 0.0)

    lo = _topk_iters(lo, 3, ts)
    hi = _topk_iters(hi, 3, ts)

    h2 = _dot(w2_ref[...], x.astype(bf16), (((0,), (0,)), ((), ())), prec)

    lo = _topk_iters(lo, 3, ts)
    hi = _topk_iters(hi, 3, ts)

    cand = jnp.concatenate(ts, axis=0)
    cts = []
    _ = _topk_iters(cand, _K, cts)
    d10 = cts[-1]

    gtf = jnp.where(A > d10, 1.0, 0.0).astype(f32)
    ones_row = jnp.ones((1, _N), f32)
    cnt_gt = _dot(ones_row, gtf, (((1,), (0,)), ((), ())), prec)
    is_common = jnp.all(cnt_gt <= jnp.float32(_K - 1))
    thresh, keep_at = jax.lax.cond(
        is_common,
        lambda: (d10, _K - cnt_gt),
        lambda: _slow_tie_state(A))

    eq = (A == thresh)
    gt = A > thresh
    ri = jax.lax.broadcasted_iota(jnp.int32, (_N, _N), 0)
    ci = jax.lax.broadcasted_iota(jnp.int32, (_N, _N), 1)
    L = (ci <= ri).astype(f32)
    cum = _dot(L, eq.astype(f32), (((1,), (0,)), ((), ())), prec)
    keep = jnp.logical_or(gt, jnp.logical_and(eq, cum <= keep_at))

    y = _dot(h2.astype(bf16), ah16, (((1,), (0,)), ((), ())), prec) \
        + b2_ref[...]
    out_ref[0] = y + v_p

    eye = (ri == ci).astype(f32)
    amt = jnp.where(keep, A, 0.0) + eye

    deg_row = jnp.sum(amt, axis=0, keepdims=True)
    ones_col = jnp.ones((_N, 1), f32)
    deg_col = _dot(amt, ones_col, (((0,), (0,)), ((), ())), prec)
    dr = jax.lax.rsqrt(deg_row)
    dr = jnp.where(jnp.isinf(dr), 0.0, dr)
    dc = jax.lax.rsqrt(deg_col)
    dc = jnp.where(jnp.isinf(dc), 0.0, dc)
    scr_ref[slot] = (dc * amt * dr).astype(bf16)


def kernel(v, g_w, g_b, gc1_w, gc1_b, gc2_w, gc2_b):
    f32 = jnp.float32
    w2_16 = gc2_w.astype(jnp.bfloat16)
    g_b2 = g_b.reshape(_C, 1)
    b1 = gc1_b.reshape(_NHID, 1)
    b2 = gc2_b.reshape(_NCLASS, 1)

    m, c1 = pl.pallas_call(
        _fold_body,
        out_shape=(jax.ShapeDtypeStruct((_NHID, _C), jnp.bfloat16),
                   jax.ShapeDtypeStruct((_NHID, 1), f32)),
    )(gc1_w, g_w, g_b2)

    full = lambda shape: pl.BlockSpec(shape, lambda b: (0,) * len(shape))
    return pl.pallas_call(
        _gcn_body,
        grid=(_B + 1,),
        in_specs=[
            pl.BlockSpec((1, _C, _N),
                         lambda b: (jnp.minimum(b, _B - 1), 0, 0)),
            pl.BlockSpec((1, _C, _N),
                         lambda b: (jnp.maximum(b - 1, 0), 0, 0)),
            full((_NHID, _C)),
            full((_NHID, 1)),
            full((_NHID, 1)),
            full((_NHID, _NCLASS)),
            full((_NCLASS, 1)),
        ],
        out_specs=pl.BlockSpec((1, _NCLASS, _N),
                               lambda b: (jnp.maximum(b - 1, 0), 0, 0)),
        out_shape=jax.ShapeDtypeStruct((_B, _NCLASS, _N), f32),
        scratch_shapes=[pltpu.VMEM((2, _N, _N), jnp.bfloat16)],
        compiler_params=pltpu.CompilerParams(
            dimension_semantics=("arbitrary",)),
    )(v, v, m, c1, b1, w2_16, b2)


# speedup vs baseline: 1.0087x; 1.0087x over previous
import jax
import jax.numpy as jnp
from jax.experimental import pallas as pl
from jax.experimental.pallas import tpu as pltpu

_B, _C, _N = 64, 1024, 256
_NHID, _NCLASS, _K = 1024, 1024, 10


def _dot(a, b, dims, prec):
    return jax.lax.dot_general(a, b, dims,
                               preferred_element_type=jnp.float32,
                               precision=prec)


def _fold_body(w1_ref, g_w_ref, g_b_ref, m_ref, c1_ref):
    prec = jax.lax.Precision.DEFAULT
    m_ref[...] = _dot(w1_ref[...], g_w_ref[...],
                      (((0,), (0,)), ((), ())), prec).astype(jnp.bfloat16)
    c1_ref[...] = _dot(w1_ref[...], g_b_ref[...],
                       (((0,), (0,)), ((), ())), prec)


def _topk_iters(cur, n_iters, ts):
    neg = jnp.float32(-jnp.inf)
    for _ in range(n_iters):
        t = jnp.max(cur, axis=0, keepdims=True)
        cur = jnp.where(cur == t, neg, cur)
        ts.append(t)
    return cur


def _slow_tie_state(A):
    f32 = jnp.float32
    neg = jnp.float32(-jnp.inf)
    cur = A
    total = jnp.zeros((1, _N), f32)
    thresh = jnp.zeros((1, _N), f32)
    keep_at = jnp.zeros((1, _N), f32)
    for _ in range(_K):
        t = jnp.max(cur, axis=0, keepdims=True)
        c = jnp.sum((cur == t).astype(f32), axis=0, keepdims=True)
        done_now = jnp.logical_and(total < _K, total + c >= _K)
        thresh = jnp.where(done_now, t, thresh)
        keep_at = jnp.where(done_now, _K - total, keep_at)
        total = total + c
        cur = jnp.where(cur == t, neg, cur)
    return thresh, keep_at


def _gcn_body(vc_ref, vp_ref, m_ref, c1_ref, b1_ref, w2_ref, b2_ref,
              out_ref, scr_ref):
    f32 = jnp.float32
    prec = jax.lax.Precision.DEFAULT
    bf16 = jnp.bfloat16
    b = pl.program_id(0)
    slot = jax.lax.rem(b, 2)

    v_c = vc_ref[0]
    v_p = vp_ref[0]
    ah16 = scr_ref[jax.lax.rem(b + 1, 2)]

    A = _dot(v_c, v_c, (((0,), (0,)), ((), ())), prec)
    h1 = _dot(m_ref[...], v_p.astype(bf16),
              (((1,), (0,)), ((), ())), prec) + c1_ref[...]

    ts = []
    cur = _topk_iters(A, 4, ts)

    x = jnp.maximum(
        _dot(h1.astype(bf16), ah16, (((1,), (0,)), ((), ())), prec)
        + b1_ref[...], 0.0)

    cur = _topk_iters(cur, 3, ts)

    h2 = _dot(w2_ref[...], x.astype(bf16), (((0,), (0,)), ((), ())), prec)

    cur = _topk_iters(cur, 3, ts)
    d10 = ts[-1]

    gtf = jnp.where(A > d10, 1.0, 0.0).astype(f32)
    ones_row = jnp.ones((1, _N), f32)
    cnt_gt = _dot(ones_row, gtf, (((1,), (0,)), ((), ())), prec)
    is_common = jnp.all(cnt_gt <= jnp.float32(_K - 1))
    thresh, keep_at = jax.lax.cond(
        is_common,
        lambda: (d10, _K - cnt_gt),
        lambda: _slow_tie_state(A))

    eq = (A == thresh)
    gt = A > thresh
    ri = jax.lax.broadcasted_iota(jnp.int32, (_N, _N), 0)
    ci = jax.lax.broadcasted_iota(jnp.int32, (_N, _N), 1)
    L = (ci <= ri).astype(f32)
    cum = _dot(L, eq.astype(f32), (((1,), (0,)), ((), ())), prec)
    keep = jnp.logical_or(gt, jnp.logical_and(eq, cum <= keep_at))

    y = _dot(h2.astype(bf16), ah16, (((1,), (0,)), ((), ())), prec) \
        + b2_ref[...]
    out_ref[0] = y + v_p

    eye = (ri == ci).astype(f32)
    amt = jnp.where(keep, A, 0.0) + eye

    deg_row = jnp.sum(amt, axis=0, keepdims=True)
    ones_col = jnp.ones((_N, 1), f32)
    deg_col = _dot(amt, ones_col, (((0,), (0,)), ((), ())), prec)
    dr = jax.lax.rsqrt(deg_row)
    dr = jnp.where(jnp.isinf(dr), 0.0, dr)
    dc = jax.lax.rsqrt(deg_col)
    dc = jnp.where(jnp.isinf(dc), 0.0, dc)
    scr_ref[slot] = (dc * amt * dr).astype(bf16)


def kernel(v, g_w, g_b, gc1_w, gc1_b, gc2_w, gc2_b):
    f32 = jnp.float32
    w2_16 = gc2_w.astype(jnp.bfloat16)
    g_b2 = g_b.reshape(_C, 1)
    b1 = gc1_b.reshape(_NHID, 1)
    b2 = gc2_b.reshape(_NCLASS, 1)

    m, c1 = pl.pallas_call(
        _fold_body,
        out_shape=(jax.ShapeDtypeStruct((_NHID, _C), jnp.bfloat16),
                   jax.ShapeDtypeStruct((_NHID, 1), f32)),
    )(gc1_w, g_w, g_b2)

    full = lambda shape: pl.BlockSpec(shape, lambda b: (0,) * len(shape))
    return pl.pallas_call(
        _gcn_body,
        grid=(_B + 1,),
        in_specs=[
            pl.BlockSpec((1, _C, _N),
                         lambda b: (jnp.minimum(b, _B - 1), 0, 0)),
            pl.BlockSpec((1, _C, _N),
                         lambda b: (jnp.maximum(b - 1, 0), 0, 0)),
            full((_NHID, _C)),
            full((_NHID, 1)),
            full((_NHID, 1)),
            full((_NHID, _NCLASS)),
            full((_NCLASS, 1)),
        ],
        out_specs=pl.BlockSpec((1, _NCLASS, _N),
                               lambda b: (jnp.maximum(b - 1, 0), 0, 0)),
        out_shape=jax.ShapeDtypeStruct((_B, _NCLASS, _N), f32),
        scratch_shapes=[pltpu.VMEM((2, _N, _N), jnp.bfloat16)],
        compiler_params=pltpu.CompilerParams(
            dimension_semantics=("arbitrary",)),
    )(v, v, m, c1, b1, w2_16, b2)
